# baseline (speedup 1.0000x reference)
import functools

import jax
import jax.numpy as jnp
from jax import lax
from jax.experimental import pallas as pl
from jax.experimental.pallas import tpu as pltpu
from jax.experimental.pallas import tpu_sc as plsc

R = 128
N = 32768
L = 16
NC = 2
NS = 16
NW = NC * NS
ROWS_PER_W = R // NW
ACCS = 8
STEPS = N // (ACCS * L)

_INT_MAX = 2**31 - 1


@functools.partial(
    pl.kernel,
    mesh=plsc.VectorSubcoreMesh(core_axis_name="c", subcore_axis_name="s"),
    out_type=jax.ShapeDtypeStruct((NW, L), jnp.int32),
    scratch_types=[
        pltpu.VMEM((2, N), jnp.float32),
        pltpu.VMEM((L,), jnp.int32),
        pltpu.VMEM((L,), jnp.float32),
        pltpu.VMEM((L,), jnp.int32),
        pltpu.SemaphoreType.DMA,
        pltpu.SemaphoreType.DMA,
    ],
    compiler_params=pltpu.CompilerParams(
        needs_layout_passes=False, skip_device_barrier=True
    ),
)
def _argmin_sc(x_hbm, out_hbm, buf, outbuf, redv, redi, sem0, sem1):
    wid = lax.axis_index("s") * NC + lax.axis_index("c")
    base_iota = lax.iota(jnp.int32, L)
    sems = (sem0, sem1)

    row0 = wid * ROWS_PER_W
    pending = pltpu.async_copy(x_hbm.at[row0], buf.at[0], sems[0])

    results = jnp.zeros((L,), dtype=jnp.int32)
    for r in range(ROWS_PER_W):
        slot = r % 2
        pending.wait()
        if r + 1 < ROWS_PER_W:
            nxt = (r + 1) % 2
            pending = pltpu.async_copy(
                x_hbm.at[row0 + r + 1], buf.at[nxt], sems[nxt]
            )

        mv0 = tuple(
            jnp.full((L,), jnp.inf, dtype=jnp.float32) for _ in range(ACCS)
        )
        mt0 = tuple(jnp.zeros((L,), dtype=jnp.int32) for _ in range(ACCS))

        @plsc.parallel_loop(0, STEPS, 1, unroll=8, carry=(mv0, mt0))
        def _scan(t, carry):
            mvs, mts = carry
            tb = jnp.full((L,), t, dtype=jnp.int32)
            new_mvs = []
            new_mts = []
            for k in range(ACCS):
                v = buf[slot, pl.ds(t * (ACCS * L) + k * L, L)]
                m = v < mvs[k]
                new_mvs.append(jnp.where(m, v, mvs[k]))
                new_mts.append(jnp.where(m, tb, mts[k]))
            return tuple(new_mvs), tuple(new_mts)

        mvs, mts = _scan
        mv = mvs[0]
        mi = mts[0] * (ACCS * L) + base_iota
        for k in range(1, ACCS):
            fi = mts[k] * (ACCS * L) + (k * L + base_iota)
            take = (mvs[k] < mv) | ((mvs[k] == mv) & (fi < mi))
            mv = jnp.where(take, mvs[k], mv)
            mi = jnp.where(take, fi, mi)

        for sh in (8, 4, 2, 1):
            redv[...] = mv
            redi[...] = mi
            perm = base_iota ^ sh
            ov = plsc.load_gather(redv, [perm])
            oi = plsc.load_gather(redi, [perm])
            take = (ov < mv) | ((ov == mv) & (oi < mi))
            mv = jnp.where(take, ov, mv)
            mi = jnp.where(take, oi, mi)
        results = jnp.where(base_iota == r, mi, results)

    outbuf[...] = results
    pltpu.sync_copy(outbuf, out_hbm.at[wid])


def kernel(x):
    staged = _argmin_sc(x)
    return staged[:, :ROWS_PER_W].reshape(R, 1)


# speedup vs baseline: 1.3292x; 1.3292x over previous
import functools

import jax
import jax.numpy as jnp
from jax import lax
from jax.experimental import pallas as pl
from jax.experimental.pallas import tpu as pltpu
from jax.experimental.pallas import tpu_sc as plsc

R = 128
N = 32768
L = 16
NC = 2
NS = 16
NW = NC * NS
ROWS_PER_W = R // NW
ACCS = 8
STEPS = N // (ACCS * L)

_INT_MAX = 2**31 - 1


@functools.partial(
    pl.kernel,
    mesh=plsc.VectorSubcoreMesh(core_axis_name="c", subcore_axis_name="s"),
    out_type=jax.ShapeDtypeStruct((NW, L), jnp.int32),
    scratch_types=[
        pltpu.VMEM((2, N), jnp.float32),
        pltpu.VMEM((L,), jnp.int32),
        pltpu.VMEM((L,), jnp.float32),
        pltpu.VMEM((L,), jnp.int32),
        pltpu.SemaphoreType.DMA,
        pltpu.SemaphoreType.DMA,
    ],
    compiler_params=pltpu.CompilerParams(
        needs_layout_passes=False, skip_device_barrier=True
    ),
)
def _argmin_sc(x_hbm, out_hbm, buf, outbuf, redv, redi, sem0, sem1):
    wid = lax.axis_index("s") * NC + lax.axis_index("c")
    base_iota = lax.iota(jnp.int32, L)
    sems = (sem0, sem1)

    row0 = wid * ROWS_PER_W
    pending = pltpu.async_copy(x_hbm.at[row0], buf.at[0], sems[0])

    results = jnp.zeros((L,), dtype=jnp.int32)
    for r in range(ROWS_PER_W):
        slot = r % 2
        pending.wait()
        if r + 1 < ROWS_PER_W:
            nxt = (r + 1) % 2
            pending = pltpu.async_copy(
                x_hbm.at[row0 + r + 1], buf.at[nxt], sems[nxt]
            )

        mv0 = tuple(
            jnp.full((L,), jnp.inf, dtype=jnp.float32) for _ in range(ACCS)
        )
        mt0 = tuple(jnp.zeros((L,), dtype=jnp.int32) for _ in range(ACCS))

        @plsc.parallel_loop(0, STEPS, 1, unroll=4, carry=(mv0, mt0))
        def _scan(t, carry):
            mvs, mts = carry
            tb = jnp.full((L,), t, dtype=jnp.int32)
            new_mvs = []
            new_mts = []
            for k in range(ACCS):
                v = buf[slot, pl.ds(t * (ACCS * L) + k * L, L)]
                m = v < mvs[k]
                new_mvs.append(jnp.where(m, v, mvs[k]))
                new_mts.append(jnp.where(m, tb, mts[k]))
            return tuple(new_mvs), tuple(new_mts)

        mvs, mts = _scan
        mv = mvs[0]
        mi = mts[0] * (ACCS * L) + base_iota
        for k in range(1, ACCS):
            fi = mts[k] * (ACCS * L) + (k * L + base_iota)
            take = (mvs[k] < mv) | ((mvs[k] == mv) & (fi < mi))
            mv = jnp.where(take, mvs[k], mv)
            mi = jnp.where(take, fi, mi)

        for sh in (8, 4, 2, 1):
            redv[...] = mv
            redi[...] = mi
            perm = base_iota ^ sh
            ov = plsc.load_gather(redv, [perm])
            oi = plsc.load_gather(redi, [perm])
            take = (ov < mv) | ((ov == mv) & (oi < mi))
            mv = jnp.where(take, ov, mv)
            mi = jnp.where(take, oi, mi)
        results = jnp.where(base_iota == r, mi, results)

    outbuf[...] = results
    pltpu.sync_copy(outbuf, out_hbm.at[wid])


def kernel(x):
    staged = _argmin_sc(x)
    return staged[:, :ROWS_PER_W].reshape(R, 1)


# speedup vs baseline: 1.3357x; 1.0049x over previous
import functools

import jax
import jax.numpy as jnp
from jax import lax
from jax.experimental import pallas as pl
from jax.experimental.pallas import tpu as pltpu
from jax.experimental.pallas import tpu_sc as plsc

R = 128
N = 32768
L = 16
NC = 2
NS = 16
NW = NC * NS
ROWS_PER_W = R // NW
ACCS = 8
STEPS = N // (ACCS * L)


@functools.partial(
    pl.kernel,
    mesh=plsc.VectorSubcoreMesh(core_axis_name="c", subcore_axis_name="s"),
    out_type=jax.ShapeDtypeStruct((NW, L), jnp.int32),
    scratch_types=[
        pltpu.VMEM((2, N), jnp.float32),
        pltpu.VMEM((L,), jnp.int32),
        pltpu.VMEM((L,), jnp.float32),
        pltpu.VMEM((L,), jnp.int32),
        pltpu.VMEM((ROWS_PER_W * ACCS * L,), jnp.float32),
        pltpu.VMEM((ROWS_PER_W * ACCS * L,), jnp.int32),
        pltpu.SemaphoreType.DMA,
        pltpu.SemaphoreType.DMA,
    ],
    compiler_params=pltpu.CompilerParams(
        needs_layout_passes=False, skip_device_barrier=True
    ),
)
def _argmin_sc(x_hbm, out_hbm, buf, outbuf, redv, redi, accv, acct, sem0, sem1):
    wid = lax.axis_index("s") * NC + lax.axis_index("c")
    base_iota = lax.iota(jnp.int32, L)
    row0 = wid * ROWS_PER_W

    pltpu.make_async_copy(x_hbm.at[row0], buf.at[0], sem0).start()
    pltpu.make_async_copy(x_hbm.at[row0 + 1], buf.at[1], sem1).start()

    def scan_row(slot, r):
        mv0 = tuple(
            jnp.full((L,), jnp.inf, dtype=jnp.float32) for _ in range(ACCS)
        )
        mt0 = tuple(jnp.zeros((L,), dtype=jnp.int32) for _ in range(ACCS))

        @plsc.parallel_loop(0, STEPS, 1, unroll=4, carry=(mv0, mt0))
        def _scan(t, carry):
            mvs, mts = carry
            tb = jnp.full((L,), t, dtype=jnp.int32)
            new_mvs = []
            new_mts = []
            for k in range(ACCS):
                v = buf[slot, pl.ds(t * (ACCS * L) + k * L, L)]
                m = v < mvs[k]
                new_mvs.append(jnp.where(m, v, mvs[k]))
                new_mts.append(jnp.where(m, tb, mts[k]))
            return tuple(new_mvs), tuple(new_mts)

        mvs, mts = _scan
        for k in range(ACCS):
            accv[pl.ds((r * ACCS + k) * L, L)] = mvs[k]
            acct[pl.ds((r * ACCS + k) * L, L)] = mts[k]

    def jbody(j, carry):
        for slot, sem in ((0, sem0), (1, sem1)):
            r = 2 * j + slot
            pltpu.make_async_copy(x_hbm.at[row0], buf.at[slot], sem).wait()
            scan_row(slot, r)

            @pl.when(j == 0)
            def _prefetch():
                pltpu.make_async_copy(
                    x_hbm.at[row0 + r + 2], buf.at[slot], sem
                ).start()

        return carry

    lax.fori_loop(0, ROWS_PER_W // 2, jbody, 0)

    def mbody(r, results):
        mv = accv[pl.ds(r * ACCS * L, L)]
        mi = acct[pl.ds(r * ACCS * L, L)] * (ACCS * L) + base_iota
        for k in range(1, ACCS):
            av = accv[pl.ds((r * ACCS + k) * L, L)]
            fi = acct[pl.ds((r * ACCS + k) * L, L)] * (ACCS * L) + (
                k * L + base_iota
            )
            take = (av < mv) | ((av == mv) & (fi < mi))
            mv = jnp.where(take, av, mv)
            mi = jnp.where(take, fi, mi)

        for sh in (8, 4, 2, 1):
            redv[...] = mv
            redi[...] = mi
            perm = base_iota ^ sh
            ov = plsc.load_gather(redv, [perm])
            oi = plsc.load_gather(redi, [perm])
            take = (ov < mv) | ((ov == mv) & (oi < mi))
            mv = jnp.where(take, ov, mv)
            mi = jnp.where(take, oi, mi)
        return jnp.where(base_iota == r, mi, results)

    results = lax.fori_loop(
        0, ROWS_PER_W, mbody, jnp.zeros((L,), dtype=jnp.int32)
    )

    outbuf[...] = results
    pltpu.sync_copy(outbuf, out_hbm.at[wid])


def kernel(x):
    staged = _argmin_sc(x)
    return staged[:, :ROWS_PER_W].reshape(R, 1)


# speedup vs baseline: 1.4592x; 1.0925x over previous
import functools

import jax
import jax.numpy as jnp
from jax import lax
from jax.experimental import pallas as pl
from jax.experimental.pallas import tpu as pltpu
from jax.experimental.pallas import tpu_sc as plsc

R = 128
N = 32768
L = 16
NC = 2
NS = 16
NW = NC * NS
ACCS = 8
STEPS = N // (ACCS * L)

TC_ROW0 = NW
TC_BLOCK = 8
TC_CHUNK = 1024
_INT_MAX = 2**31 - 1


@functools.partial(
    pl.kernel,
    mesh=plsc.VectorSubcoreMesh(core_axis_name="c", subcore_axis_name="s"),
    out_type=jax.ShapeDtypeStruct((NW, L), jnp.int32),
    scratch_types=[
        pltpu.VMEM((N,), jnp.float32),
        pltpu.VMEM((L,), jnp.int32),
        pltpu.VMEM((L,), jnp.float32),
        pltpu.VMEM((L,), jnp.int32),
    ],
    compiler_params=pltpu.CompilerParams(
        needs_layout_passes=False, skip_device_barrier=True
    ),
)
def _argmin_sc(x_hbm, out_hbm, buf, outbuf, redv, redi):
    wid = lax.axis_index("s") * NC + lax.axis_index("c")
    base_iota = lax.iota(jnp.int32, L)

    pltpu.sync_copy(x_hbm.at[wid], buf)

    mv0 = tuple(jnp.full((L,), jnp.inf, dtype=jnp.float32) for _ in range(ACCS))
    mt0 = tuple(jnp.zeros((L,), dtype=jnp.int32) for _ in range(ACCS))

    @plsc.parallel_loop(0, STEPS, 1, unroll=4, carry=(mv0, mt0))
    def _scan(t, carry):
        mvs, mts = carry
        tb = jnp.full((L,), t, dtype=jnp.int32)
        new_mvs = []
        new_mts = []
        for k in range(ACCS):
            v = buf[pl.ds(t * (ACCS * L) + k * L, L)]
            m = v < mvs[k]
            new_mvs.append(jnp.where(m, v, mvs[k]))
            new_mts.append(jnp.where(m, tb, mts[k]))
        return tuple(new_mvs), tuple(new_mts)

    mvs, mts = _scan
    mv = mvs[0]
    mi = mts[0] * (ACCS * L) + base_iota
    for k in range(1, ACCS):
        fi = mts[k] * (ACCS * L) + (k * L + base_iota)
        take = (mvs[k] < mv) | ((mvs[k] == mv) & (fi < mi))
        mv = jnp.where(take, mvs[k], mv)
        mi = jnp.where(take, fi, mi)

    for sh in (8, 4, 2, 1):
        redv[...] = mv
        redi[...] = mi
        perm = base_iota ^ sh
        ov = plsc.load_gather(redv, [perm])
        oi = plsc.load_gather(redi, [perm])
        take = (ov < mv) | ((ov == mv) & (oi < mi))
        mv = jnp.where(take, ov, mv)
        mi = jnp.where(take, oi, mi)

    outbuf[...] = mi
    pltpu.sync_copy(outbuf, out_hbm.at[wid])


def _tc_body(x_ref, o_ref):
    bmv0 = jnp.full((TC_BLOCK, TC_CHUNK), jnp.inf, dtype=jnp.float32)
    bci0 = jnp.zeros((TC_BLOCK, TC_CHUNK), dtype=jnp.int32)

    def chunk(c, carry):
        bmv, bci = carry
        v = x_ref[:, pl.ds(c * TC_CHUNK, TC_CHUNK)]
        m = v < bmv
        bmv = jnp.where(m, v, bmv)
        bci = jnp.where(m, c, bci)
        return bmv, bci

    bmv, bci = lax.fori_loop(0, N // TC_CHUNK, chunk, (bmv0, bci0))

    rowmin = jnp.min(bmv, axis=1, keepdims=True)
    pos = lax.broadcasted_iota(jnp.int32, (TC_BLOCK, TC_CHUNK), 1)
    flat = bci * TC_CHUNK + pos
    cand = jnp.where(bmv == rowmin, flat, _INT_MAX)
    o_ref[...] = jnp.min(cand, axis=1, keepdims=True)


_argmin_tc = pl.pallas_call(
    _tc_body,
    grid=((R - TC_ROW0) // TC_BLOCK,),
    in_specs=[
        pl.BlockSpec((TC_BLOCK, N), lambda i: (i + TC_ROW0 // TC_BLOCK, 0))
    ],
    out_specs=pl.BlockSpec((TC_BLOCK, 1), lambda i: (i, 0)),
    out_shape=jax.ShapeDtypeStruct((R - TC_ROW0, 1), jnp.int32),
)


def kernel(x):
    sc_part = _argmin_sc(x)
    tc_part = _argmin_tc(x)
    return jnp.concatenate([sc_part[:, :1], tc_part], axis=0)


# speedup vs baseline: 1.4751x; 1.0109x over previous
import functools

import jax
import jax.numpy as jnp
from jax import lax
from jax.experimental import pallas as pl
from jax.experimental.pallas import tpu as pltpu
from jax.experimental.pallas import tpu_sc as plsc

R = 128
N = 32768
L = 16
NC = 2
NS = 16
NW = NC * NS
ACCS = 8
STEPS = N // (ACCS * L)

TC_ROW0 = NW
TC_BLOCK = 8
TC_CHUNK = 1024
_INT_MAX = 2**31 - 1


@functools.partial(
    pl.kernel,
    mesh=plsc.VectorSubcoreMesh(core_axis_name="c", subcore_axis_name="s"),
    out_type=jax.ShapeDtypeStruct((NW, L), jnp.int32),
    scratch_types=[
        pltpu.VMEM((N,), jnp.float32),
        pltpu.VMEM((L,), jnp.int32),
        pltpu.VMEM((L,), jnp.float32),
        pltpu.VMEM((L,), jnp.int32),
    ],
    compiler_params=pltpu.CompilerParams(
        needs_layout_passes=False, skip_device_barrier=True
    ),
)
def _argmin_sc(x_hbm, out_hbm, buf, outbuf, redv, redi):
    wid = lax.axis_index("s") * NC + lax.axis_index("c")
    base_iota = lax.iota(jnp.int32, L)

    pltpu.sync_copy(x_hbm.at[wid], buf)

    mv0 = tuple(jnp.full((L,), jnp.inf, dtype=jnp.float32) for _ in range(ACCS))
    mt0 = tuple(jnp.zeros((L,), dtype=jnp.int32) for _ in range(ACCS))

    @plsc.parallel_loop(0, STEPS, 1, unroll=4, carry=(mv0, mt0))
    def _scan(t, carry):
        mvs, mts = carry
        tb = jnp.full((L,), t, dtype=jnp.int32)
        new_mvs = []
        new_mts = []
        for k in range(ACCS):
            v = buf[pl.ds(t * (ACCS * L) + k * L, L)]
            m = v < mvs[k]
            new_mvs.append(jnp.where(m, v, mvs[k]))
            new_mts.append(jnp.where(m, tb, mts[k]))
        return tuple(new_mvs), tuple(new_mts)

    mvs, mts = _scan
    mv = mvs[0]
    mi = mts[0] * (ACCS * L) + base_iota
    for k in range(1, ACCS):
        fi = mts[k] * (ACCS * L) + (k * L + base_iota)
        take = (mvs[k] < mv) | ((mvs[k] == mv) & (fi < mi))
        mv = jnp.where(take, mvs[k], mv)
        mi = jnp.where(take, fi, mi)

    for sh in (8, 4, 2, 1):
        redv[...] = mv
        redi[...] = mi
        perm = base_iota ^ sh
        ov = plsc.load_gather(redv, [perm])
        oi = plsc.load_gather(redi, [perm])
        take = (ov < mv) | ((ov == mv) & (oi < mi))
        mv = jnp.where(take, ov, mv)
        mi = jnp.where(take, oi, mi)

    outbuf[...] = mi
    pltpu.sync_copy(outbuf, out_hbm.at[wid])


def _tc_body(x_ref, o_ref):
    bmv = jnp.full((TC_BLOCK, TC_CHUNK), jnp.inf, dtype=jnp.float32)
    bci = jnp.zeros((TC_BLOCK, TC_CHUNK), dtype=jnp.int32)
    for c in range(N // TC_CHUNK):
        v = x_ref[:, pl.ds(c * TC_CHUNK, TC_CHUNK)]
        m = v < bmv
        bmv = jnp.where(m, v, bmv)
        bci = jnp.where(m, jnp.int32(c), bci)

    rowmin = jnp.min(bmv, axis=1, keepdims=True)
    pos = lax.broadcasted_iota(jnp.int32, (TC_BLOCK, TC_CHUNK), 1)
    flat = bci * TC_CHUNK + pos
    cand = jnp.where(bmv == rowmin, flat, _INT_MAX)
    o_ref[...] = jnp.min(cand, axis=1, keepdims=True)


_argmin_tc = pl.pallas_call(
    _tc_body,
    grid=((R - TC_ROW0) // TC_BLOCK,),
    in_specs=[
        pl.BlockSpec((TC_BLOCK, N), lambda i: (i + TC_ROW0 // TC_BLOCK, 0))
    ],
    out_specs=pl.BlockSpec((TC_BLOCK, 1), lambda i: (i, 0)),
    out_shape=jax.ShapeDtypeStruct((R - TC_ROW0, 1), jnp.int32),
)


def _asm_body(staged_ref, tc_ref, o_ref):
    o_ref[pl.ds(0, NW), :] = staged_ref[:, :1]
    o_ref[pl.ds(NW, R - TC_ROW0), :] = tc_ref[...]


_assemble = pl.pallas_call(
    _asm_body,
    out_shape=jax.ShapeDtypeStruct((R, 1), jnp.int32),
)


def kernel(x):
    tc_part = _argmin_tc(x)
    sc_part = _argmin_sc(x)
    return _assemble(sc_part, tc_part)


# speedup vs baseline: 1.5260x; 1.0345x over previous
import functools

import jax
import jax.numpy as jnp
from jax import lax
from jax.experimental import pallas as pl
from jax.experimental.pallas import tpu as pltpu
from jax.experimental.pallas import tpu_sc as plsc

R = 128
N = 32768
L = 16
NC = 2
NS = 16
NW = NC * NS
SC_RPW = 2
SC_ROWS = NW * SC_RPW
ACCS = 8
STEPS = N // (ACCS * L)

TC_BLOCK = 8
TC_CHUNK = 1024
_INT_MAX = 2**31 - 1


@functools.partial(
    pl.kernel,
    mesh=plsc.VectorSubcoreMesh(core_axis_name="c", subcore_axis_name="s"),
    out_type=jax.ShapeDtypeStruct((NW, L), jnp.int32),
    scratch_types=[
        pltpu.VMEM((2, N), jnp.float32),
        pltpu.VMEM((L,), jnp.int32),
        pltpu.VMEM((L,), jnp.float32),
        pltpu.VMEM((L,), jnp.int32),
        pltpu.SemaphoreType.DMA,
        pltpu.SemaphoreType.DMA,
    ],
    compiler_params=pltpu.CompilerParams(
        needs_layout_passes=False, skip_device_barrier=True
    ),
)
def _argmin_sc(x_hbm, out_hbm, buf, outbuf, redv, redi, sem0, sem1):
    wid = lax.axis_index("s") * NC + lax.axis_index("c")
    base_iota = lax.iota(jnp.int32, L)
    row0 = wid * SC_RPW

    pltpu.make_async_copy(x_hbm.at[row0], buf.at[0], sem0).start()
    pltpu.make_async_copy(x_hbm.at[row0 + 1], buf.at[1], sem1).start()

    def scan_row(slot):
        mv0 = tuple(
            jnp.full((L,), jnp.inf, dtype=jnp.float32) for _ in range(ACCS)
        )
        mt0 = tuple(jnp.zeros((L,), dtype=jnp.int32) for _ in range(ACCS))

        @plsc.parallel_loop(0, STEPS, 1, unroll=4, carry=(mv0, mt0))
        def _scan(t, carry):
            mvs, mts = carry
            tb = jnp.full((L,), t, dtype=jnp.int32)
            new_mvs = []
            new_mts = []
            for k in range(ACCS):
                v = buf[slot, pl.ds(t * (ACCS * L) + k * L, L)]
                m = v < mvs[k]
                new_mvs.append(jnp.where(m, v, mvs[k]))
                new_mts.append(jnp.where(m, tb, mts[k]))
            return tuple(new_mvs), tuple(new_mts)

        mvs, mts = _scan
        mv = mvs[0]
        mi = mts[0] * (ACCS * L) + base_iota
        for k in range(1, ACCS):
            fi = mts[k] * (ACCS * L) + (k * L + base_iota)
            take = (mvs[k] < mv) | ((mvs[k] == mv) & (fi < mi))
            mv = jnp.where(take, mvs[k], mv)
            mi = jnp.where(take, fi, mi)

        for sh in (8, 4, 2, 1):
            redv[...] = mv
            redi[...] = mi
            perm = base_iota ^ sh
            ov = plsc.load_gather(redv, [perm])
            oi = plsc.load_gather(redi, [perm])
            take = (ov < mv) | ((ov == mv) & (oi < mi))
            mv = jnp.where(take, ov, mv)
            mi = jnp.where(take, oi, mi)
        return mi

    pltpu.make_async_copy(x_hbm.at[row0], buf.at[0], sem0).wait()
    mi_a = scan_row(0)
    pltpu.make_async_copy(x_hbm.at[row0 + 1], buf.at[1], sem1).wait()
    mi_b = scan_row(1)

    outbuf[...] = jnp.where(base_iota == 0, mi_a, mi_b)
    pltpu.sync_copy(outbuf, out_hbm.at[wid])


def _tc_body(x_ref, o_ref):
    bmv = jnp.full((TC_BLOCK, TC_CHUNK), jnp.inf, dtype=jnp.float32)
    bci = jnp.zeros((TC_BLOCK, TC_CHUNK), dtype=jnp.int32)
    for c in range(N // TC_CHUNK):
        v = x_ref[:, pl.ds(c * TC_CHUNK, TC_CHUNK)]
        m = v < bmv
        bmv = jnp.where(m, v, bmv)
        bci = jnp.where(m, jnp.int32(c), bci)

    rowmin = jnp.min(bmv, axis=1, keepdims=True)
    pos = lax.broadcasted_iota(jnp.int32, (TC_BLOCK, TC_CHUNK), 1)
    flat = bci * TC_CHUNK + pos
    cand = jnp.where(bmv == rowmin, flat, _INT_MAX)
    o_ref[...] = jnp.min(cand, axis=1, keepdims=True)


_argmin_tc = pl.pallas_call(
    _tc_body,
    grid=((R - SC_ROWS) // TC_BLOCK,),
    in_specs=[
        pl.BlockSpec((TC_BLOCK, N), lambda i: (i + SC_ROWS // TC_BLOCK, 0))
    ],
    out_specs=pl.BlockSpec((TC_BLOCK, 1), lambda i: (i + SC_ROWS // TC_BLOCK, 0)),
    out_shape=jax.ShapeDtypeStruct((R, 1), jnp.int32),
)


def kernel(x):
    tc_full = _argmin_tc(x)
    sc_part = _argmin_sc(x)
    sc_rows = sc_part[:, :SC_RPW].reshape(SC_ROWS, 1)
    return lax.dynamic_update_slice(tc_full, sc_rows, (0, 0))


# speedup vs baseline: 1.5337x; 1.0050x over previous
import functools

import jax
import jax.numpy as jnp
from jax import lax
from jax.experimental import pallas as pl
from jax.experimental.pallas import tpu as pltpu
from jax.experimental.pallas import tpu_sc as plsc

R = 128
N = 32768
L = 16
NC = 2
NS = 16
NW = NC * NS
SC_RPW = 2
SC_ROWS = NW * SC_RPW
ACCS = 8
STEPS = N // (ACCS * L)

TC_BLOCK = 8
TC_CHUNK = 1024
_INT_MAX = 2**31 - 1


@functools.partial(
    pl.kernel,
    mesh=plsc.VectorSubcoreMesh(core_axis_name="c", subcore_axis_name="s"),
    out_type=jax.ShapeDtypeStruct((SC_ROWS,), jnp.int32),
    scratch_types=[
        pltpu.VMEM((2, N), jnp.float32),
        pltpu.VMEM((L,), jnp.int32),
        pltpu.VMEM((L,), jnp.float32),
        pltpu.VMEM((L,), jnp.int32),
        pltpu.VMEM_SHARED((NS, L), jnp.int32),
        pltpu.VMEM((NS, L), jnp.int32),
        pltpu.VMEM((2 * L,), jnp.int32),
        pltpu.SemaphoreType.DMA,
        pltpu.SemaphoreType.DMA,
    ],
    compiler_params=pltpu.CompilerParams(
        needs_layout_passes=False, skip_device_barrier=True
    ),
)
def _argmin_sc(
    x_hbm, out_hbm, buf, outbuf, redv, redi, shared, gbuf, outv, sem0, sem1
):
    cid = lax.axis_index("c")
    sid = lax.axis_index("s")
    wid = cid * NS + sid
    base_iota = lax.iota(jnp.int32, L)
    row0 = wid * SC_RPW

    pltpu.make_async_copy(x_hbm.at[row0], buf.at[0], sem0).start()
    pltpu.make_async_copy(x_hbm.at[row0 + 1], buf.at[1], sem1).start()

    def scan_row(slot):
        mv0 = tuple(
            jnp.full((L,), jnp.inf, dtype=jnp.float32) for _ in range(ACCS)
        )
        mt0 = tuple(jnp.zeros((L,), dtype=jnp.int32) for _ in range(ACCS))

        @plsc.parallel_loop(0, STEPS, 1, unroll=4, carry=(mv0, mt0))
        def _scan(t, carry):
            mvs, mts = carry
            tb = jnp.full((L,), t, dtype=jnp.int32)
            new_mvs = []
            new_mts = []
            for k in range(ACCS):
                v = buf[slot, pl.ds(t * (ACCS * L) + k * L, L)]
                m = v < mvs[k]
                new_mvs.append(jnp.where(m, v, mvs[k]))
                new_mts.append(jnp.where(m, tb, mts[k]))
            return tuple(new_mvs), tuple(new_mts)

        mvs, mts = _scan
        mv = mvs[0]
        mi = mts[0] * (ACCS * L) + base_iota
        for k in range(1, ACCS):
            fi = mts[k] * (ACCS * L) + (k * L + base_iota)
            take = (mvs[k] < mv) | ((mvs[k] == mv) & (fi < mi))
            mv = jnp.where(take, mvs[k], mv)
            mi = jnp.where(take, fi, mi)

        for sh in (8, 4, 2, 1):
            redv[...] = mv
            redi[...] = mi
            perm = base_iota ^ sh
            ov = plsc.load_gather(redv, [perm])
            oi = plsc.load_gather(redi, [perm])
            take = (ov < mv) | ((ov == mv) & (oi < mi))
            mv = jnp.where(take, ov, mv)
            mi = jnp.where(take, oi, mi)
        return mi

    pltpu.make_async_copy(x_hbm.at[row0], buf.at[0], sem0).wait()
    mi_a = scan_row(0)
    pltpu.make_async_copy(x_hbm.at[row0 + 1], buf.at[1], sem1).wait()
    mi_b = scan_row(1)

    outbuf[...] = jnp.where(base_iota == 0, mi_a, mi_b)
    pltpu.sync_copy(outbuf, shared.at[sid])
    plsc.subcore_barrier()

    @pl.when(sid == 0)
    def _assemble_core():
        pltpu.sync_copy(shared, gbuf)
        rows = lax.shift_right_logical(base_iota, 1)
        lanes = base_iota & 1
        outv[pl.ds(0, L)] = plsc.load_gather(gbuf, [rows, lanes])
        outv[pl.ds(L, L)] = plsc.load_gather(gbuf, [rows + 8, lanes])
        pltpu.sync_copy(outv, out_hbm.at[pl.ds(cid * (NS * SC_RPW), 2 * L)])


def _tc_body(x_ref, o_ref):
    bmv = jnp.full((TC_BLOCK, TC_CHUNK), jnp.inf, dtype=jnp.float32)
    bci = jnp.zeros((TC_BLOCK, TC_CHUNK), dtype=jnp.int32)
    for c in range(N // TC_CHUNK):
        v = x_ref[:, pl.ds(c * TC_CHUNK, TC_CHUNK)]
        m = v < bmv
        bmv = jnp.where(m, v, bmv)
        bci = jnp.where(m, jnp.int32(c), bci)

    rowmin = jnp.min(bmv, axis=1, keepdims=True)
    pos = lax.broadcasted_iota(jnp.int32, (TC_BLOCK, TC_CHUNK), 1)
    flat = bci * TC_CHUNK + pos
    cand = jnp.where(bmv == rowmin, flat, _INT_MAX)
    o_ref[...] = jnp.min(cand, axis=1, keepdims=True)


_argmin_tc = pl.pallas_call(
    _tc_body,
    grid=((R - SC_ROWS) // TC_BLOCK,),
    in_specs=[
        pl.BlockSpec((TC_BLOCK, N), lambda i: (i + SC_ROWS // TC_BLOCK, 0))
    ],
    out_specs=pl.BlockSpec((TC_BLOCK, 1), lambda i: (i + SC_ROWS // TC_BLOCK, 0)),
    out_shape=jax.ShapeDtypeStruct((R, 1), jnp.int32),
)


def kernel(x):
    tc_full = _argmin_tc(x)
    sc_rows = _argmin_sc(x)
    return lax.dynamic_update_slice(tc_full, sc_rows.reshape(SC_ROWS, 1), (0, 0))
